# Initial kernel scaffold; baseline (speedup 1.0000x reference)
import jax
import jax.numpy as jnp
from jax.experimental import pallas as pl


def kernel(x, edge_index, W1, b1, W2, b2, s):
    raise NotImplementedError("write your pallas kernel here")


# speedup vs baseline: 14.9801x; 14.9801x over previous
import functools

import jax
import jax.numpy as jnp
from jax import lax
from jax.experimental import pallas as pl
from jax.experimental.pallas import tpu as pltpu
from jax.experimental.pallas import tpu_sc as plsc

N = 10000
E = 320000
HOP = 10
D = 64
DIN = 128
NC = 2
NS = 16
NW = NC * NS
EPW = E // NW
CHUNK = 80
NCHUNK = EPW // CHUNK
NPAD = 10240
RPT = NPAD // NS
ZROWS = 128

_mesh = plsc.VectorSubcoreMesh(core_axis_name="c", subcore_axis_name="s")
_sc_params = pltpu.CompilerParams(use_tc_tiling_on_sc=False)


@functools.partial(
    pl.kernel,
    out_type=jax.ShapeDtypeStruct((NC, NPAD), jnp.float32),
    mesh=_mesh,
    compiler_params=_sc_params,
    scratch_types=[
        pltpu.VMEM_SHARED((NPAD,), jnp.float32),
        pltpu.VMEM((NCHUNK, CHUNK), jnp.int32),
        pltpu.VMEM((CHUNK,), jnp.float32),
        pltpu.VMEM((RPT,), jnp.float32),
    ],
)
def _deg_kernel(dst_hbm, zh_hbm, out_hbm, acc, didx, ones, zbuf):
    c = lax.axis_index("c")
    s = lax.axis_index("s")
    wid = c * NS + s
    pltpu.sync_copy(dst_hbm.at[wid], didx)
    pltpu.sync_copy(zh_hbm, zbuf)
    for i in range(CHUNK // 16):
        ones[pl.ds(i * 16, 16)] = jnp.ones((16,), jnp.float32)
    pltpu.sync_copy(zbuf, acc.at[pl.ds(s * RPT, RPT)])
    plsc.subcore_barrier()

    def chunk(j, carry):
        pltpu.sync_copy(ones, acc.at[didx.at[j]], add=True)
        return carry

    lax.fori_loop(0, NCHUNK, chunk, 0)
    plsc.subcore_barrier()
    pltpu.sync_copy(acc.at[pl.ds(s * RPT, RPT)],
                    out_hbm.at[c, pl.ds(s * RPT, RPT)])


@functools.partial(
    pl.kernel,
    out_type=jax.ShapeDtypeStruct((NC, NPAD, D), jnp.float32),
    mesh=_mesh,
    compiler_params=_sc_params,
    scratch_types=[
        pltpu.VMEM_SHARED((NPAD, D), jnp.float32),
        pltpu.VMEM((NCHUNK, CHUNK), jnp.int32),
        pltpu.VMEM((NCHUNK, CHUNK), jnp.int32),
        pltpu.VMEM((CHUNK, D), jnp.float32),
        pltpu.VMEM((ZROWS, D), jnp.float32),
        pltpu.SemaphoreType.DMA,
    ],
)
def _hop_kernel(src_hbm, dst_hbm, g_hbm, zv_hbm, out_hbm,
                acc, sidx, didx, rows, zbuf, sem):
    c = lax.axis_index("c")
    s = lax.axis_index("s")
    wid = c * NS + s
    pltpu.sync_copy(src_hbm.at[wid], sidx)
    pltpu.sync_copy(dst_hbm.at[wid], didx)
    pltpu.sync_copy(zv_hbm, zbuf)
    for b in range(RPT // ZROWS):
        pltpu.sync_copy(zbuf, acc.at[pl.ds(s * RPT + b * ZROWS, ZROWS)])
    plsc.subcore_barrier()

    def chunk(j, carry):
        pltpu.async_copy(g_hbm.at[sidx.at[j]], rows, sem).wait()
        pltpu.sync_copy(rows, acc.at[didx.at[j]], add=True)
        return carry

    lax.fori_loop(0, NCHUNK, chunk, 0)
    plsc.subcore_barrier()
    pltpu.sync_copy(acc.at[pl.ds(s * RPT, RPT)],
                    out_hbm.at[c, pl.ds(s * RPT, RPT)])


_MLP_B = 2000


def _mlp_body(x_ref, w1_ref, b1_ref, w2_ref, b2_ref, hp_ref,
              g0_ref, d2_ref, r_ref):
    deg = hp_ref[...][:, 0:1] + hp_ref[...][:, 1:2] + 1.0
    dinv = lax.rsqrt(deg)
    h = jnp.maximum(
        jnp.dot(x_ref[...], w1_ref[...], preferred_element_type=jnp.float32)
        + b1_ref[...], 0.0)
    h = jnp.dot(h, w2_ref[...], preferred_element_type=jnp.float32) + b2_ref[...]
    g0_ref[...] = h * dinv
    d2_ref[...] = dinv * dinv
    r_ref[...] = deg * dinv


def _mlp_call(x, W1, b1, W2, b2, hp2):
    nblk = N // _MLP_B
    return pl.pallas_call(
        _mlp_body,
        grid=(nblk,),
        in_specs=[
            pl.BlockSpec((_MLP_B, DIN), lambda i: (i, 0)),
            pl.BlockSpec((DIN, DIN), lambda i: (0, 0)),
            pl.BlockSpec((1, DIN), lambda i: (0, 0)),
            pl.BlockSpec((DIN, D), lambda i: (0, 0)),
            pl.BlockSpec((1, D), lambda i: (0, 0)),
            pl.BlockSpec((_MLP_B, 2), lambda i: (i, 0)),
        ],
        out_specs=[
            pl.BlockSpec((_MLP_B, D), lambda i: (i, 0)),
            pl.BlockSpec((_MLP_B, 1), lambda i: (i, 0)),
            pl.BlockSpec((_MLP_B, 1), lambda i: (i, 0)),
        ],
        out_shape=[
            jax.ShapeDtypeStruct((N, D), jnp.float32),
            jax.ShapeDtypeStruct((N, 1), jnp.float32),
            jax.ShapeDtypeStruct((N, 1), jnp.float32),
        ],
    )(x, W1, b1, W2, b2, hp2)


_CMB_B = 2000


def _comb_body(p_ref, g_ref, d2_ref, o_ref):
    o_ref[...] = d2_ref[...] * (p_ref[0] + p_ref[1] + g_ref[...])


def _comb_call(P, g, d2):
    nblk = N // _CMB_B
    return pl.pallas_call(
        _comb_body,
        grid=(nblk,),
        in_specs=[
            pl.BlockSpec((NC, _CMB_B, D), lambda i: (0, i, 0)),
            pl.BlockSpec((_CMB_B, D), lambda i: (i, 0)),
            pl.BlockSpec((_CMB_B, 1), lambda i: (i, 0)),
        ],
        out_specs=pl.BlockSpec((_CMB_B, D), lambda i: (i, 0)),
        out_shape=jax.ShapeDtypeStruct((N, D), jnp.float32),
    )(P, g, d2)


_POOL_B = 2000


def _pool_body(*refs):
    g_refs = refs[:HOP + 1]
    r_ref, s_ref, o_ref = refs[HOP + 1:]
    r = r_ref[...]
    sv = s_ref[...]
    acc = jnp.zeros((_POOL_B, D), jnp.float32)
    for k in range(HOP + 1):
        hk = g_refs[k][...] * r
        z = jnp.dot(hk, sv, preferred_element_type=jnp.float32)
        acc = acc + jax.nn.sigmoid(z) * hk
    o_ref[...] = acc


def _pool_call(gs, r, s):
    nblk = N // _POOL_B
    in_specs = [pl.BlockSpec((_POOL_B, D), lambda i: (i, 0))
                for _ in range(HOP + 1)]
    in_specs.append(pl.BlockSpec((_POOL_B, 1), lambda i: (i, 0)))
    in_specs.append(pl.BlockSpec((D, 1), lambda i: (0, 0)))
    return pl.pallas_call(
        _pool_body,
        grid=(nblk,),
        in_specs=in_specs,
        out_specs=pl.BlockSpec((_POOL_B, D), lambda i: (i, 0)),
        out_shape=jax.ShapeDtypeStruct((N, D), jnp.float32),
    )(*gs, r, s)


def kernel(x, edge_index, W1, b1, W2, b2, s):
    src = edge_index[0].astype(jnp.int32).reshape(NW, NCHUNK, CHUNK)
    dst = edge_index[1].astype(jnp.int32).reshape(NW, NCHUNK, CHUNK)
    zh = jnp.zeros((RPT,), jnp.float32)
    zv = jnp.zeros((ZROWS, D), jnp.float32)

    hp = _deg_kernel(dst, zh)
    hp2 = jnp.transpose(hp[:, :N])

    g, d2, r = _mlp_call(x, W1, b1.reshape(1, DIN), W2, b2.reshape(1, D), hp2)

    gs = [g]
    for _ in range(HOP):
        P = _hop_kernel(src, dst, g, zv)
        g = _comb_call(P, g, d2)
        gs.append(g)

    return _pool_call(gs, r, s)


# speedup vs baseline: 23.1977x; 1.5486x over previous
import functools

import jax
import jax.numpy as jnp
from jax import lax
from jax.experimental import pallas as pl
from jax.experimental.pallas import tpu as pltpu
from jax.experimental.pallas import tpu_sc as plsc

N = 10000
E = 320000
HOP = 10
D = 64
DIN = 128
NC = 2
NS = 16
NW = NC * NS
EPW = E // NW
CHUNK = 80
NCHUNK = EPW // CHUNK
NPAD = 10240
RPT = NPAD // NS
ZROWS = 128

_mesh = plsc.VectorSubcoreMesh(core_axis_name="c", subcore_axis_name="s")
_sc_params = pltpu.CompilerParams(use_tc_tiling_on_sc=False)


@functools.partial(
    pl.kernel,
    out_type=jax.ShapeDtypeStruct((NC, NPAD), jnp.float32),
    mesh=_mesh,
    compiler_params=_sc_params,
    scratch_types=[
        pltpu.VMEM_SHARED((NPAD,), jnp.float32),
        pltpu.VMEM((NCHUNK, CHUNK), jnp.int32),
        pltpu.VMEM((CHUNK,), jnp.float32),
        pltpu.VMEM((RPT,), jnp.float32),
    ],
)
def _deg_kernel(dst_hbm, zh_hbm, out_hbm, acc, didx, ones, zbuf):
    c = lax.axis_index("c")
    s = lax.axis_index("s")
    wid = c * NS + s
    pltpu.sync_copy(dst_hbm.at[wid], didx)
    pltpu.sync_copy(zh_hbm, zbuf)
    for i in range(CHUNK // 16):
        ones[pl.ds(i * 16, 16)] = jnp.ones((16,), jnp.float32)
    pltpu.sync_copy(zbuf, acc.at[pl.ds(s * RPT, RPT)])
    plsc.subcore_barrier()

    def chunk(j, carry):
        pltpu.sync_copy(ones, acc.at[didx.at[j]], add=True)
        return carry

    lax.fori_loop(0, NCHUNK, chunk, 0)
    plsc.subcore_barrier()
    pltpu.sync_copy(acc.at[pl.ds(s * RPT, RPT)],
                    out_hbm.at[c, pl.ds(s * RPT, RPT)])


@functools.partial(
    pl.kernel,
    out_type=jax.ShapeDtypeStruct((NC, NPAD, D), jnp.float32),
    mesh=_mesh,
    compiler_params=_sc_params,
    scratch_types=[
        pltpu.VMEM_SHARED((NPAD, D), jnp.float32),
        pltpu.VMEM((NCHUNK, CHUNK), jnp.int32),
        pltpu.VMEM((NCHUNK, CHUNK), jnp.int32),
        pltpu.VMEM((2, CHUNK, D), jnp.float32),
        pltpu.VMEM((ZROWS, D), jnp.float32),
        pltpu.SemaphoreType.DMA,
        pltpu.SemaphoreType.DMA,
    ],
)
def _hop_kernel(src_hbm, dst_hbm, g_hbm, zv_hbm, out_hbm,
                acc, sidx, didx, rows, zbuf, sem0, sem1):
    c = lax.axis_index("c")
    s = lax.axis_index("s")
    wid = c * NS + s
    pltpu.sync_copy(src_hbm.at[wid], sidx)
    pltpu.sync_copy(dst_hbm.at[wid], didx)
    pltpu.sync_copy(zv_hbm, zbuf)
    for b in range(RPT // ZROWS):
        pltpu.sync_copy(zbuf, acc.at[pl.ds(s * RPT + b * ZROWS, ZROWS)])
    plsc.subcore_barrier()

    pltpu.async_copy(g_hbm.at[sidx.at[0]], rows.at[0], sem0)

    def pair(t, carry):
        j0 = 2 * t
        pltpu.async_copy(g_hbm.at[sidx.at[j0 + 1]], rows.at[1], sem1)
        pltpu.make_async_copy(g_hbm.at[sidx.at[j0]], rows.at[0], sem0).wait()
        pltpu.sync_copy(rows.at[0], acc.at[didx.at[j0]], add=True)
        pltpu.async_copy(g_hbm.at[sidx.at[j0 + 2]], rows.at[0], sem0)
        pltpu.make_async_copy(
            g_hbm.at[sidx.at[j0 + 1]], rows.at[1], sem1).wait()
        pltpu.sync_copy(rows.at[1], acc.at[didx.at[j0 + 1]], add=True)
        return carry

    lax.fori_loop(0, (NCHUNK - 1) // 2, pair, 0)
    pltpu.make_async_copy(
        g_hbm.at[sidx.at[NCHUNK - 1]], rows.at[0], sem0).wait()
    pltpu.sync_copy(rows.at[0], acc.at[didx.at[NCHUNK - 1]], add=True)
    plsc.subcore_barrier()
    pltpu.sync_copy(acc.at[pl.ds(s * RPT, RPT)],
                    out_hbm.at[c, pl.ds(s * RPT, RPT)])


_MLP_B = 2000


def _mlp_body(x_ref, w1_ref, b1_ref, w2_ref, b2_ref, hp_ref,
              g0_ref, d2_ref, r_ref):
    deg = hp_ref[...][:, 0:1] + hp_ref[...][:, 1:2] + 1.0
    dinv = lax.rsqrt(deg)
    h = jnp.maximum(
        jnp.dot(x_ref[...], w1_ref[...], preferred_element_type=jnp.float32)
        + b1_ref[...], 0.0)
    h = jnp.dot(h, w2_ref[...], preferred_element_type=jnp.float32) + b2_ref[...]
    g0_ref[...] = h * dinv
    d2_ref[...] = dinv * dinv
    r_ref[...] = deg * dinv


def _mlp_call(x, W1, b1, W2, b2, hp2):
    nblk = N // _MLP_B
    return pl.pallas_call(
        _mlp_body,
        grid=(nblk,),
        in_specs=[
            pl.BlockSpec((_MLP_B, DIN), lambda i: (i, 0)),
            pl.BlockSpec((DIN, DIN), lambda i: (0, 0)),
            pl.BlockSpec((1, DIN), lambda i: (0, 0)),
            pl.BlockSpec((DIN, D), lambda i: (0, 0)),
            pl.BlockSpec((1, D), lambda i: (0, 0)),
            pl.BlockSpec((_MLP_B, 2), lambda i: (i, 0)),
        ],
        out_specs=[
            pl.BlockSpec((_MLP_B, D), lambda i: (i, 0)),
            pl.BlockSpec((_MLP_B, 1), lambda i: (i, 0)),
            pl.BlockSpec((_MLP_B, 1), lambda i: (i, 0)),
        ],
        out_shape=[
            jax.ShapeDtypeStruct((N, D), jnp.float32),
            jax.ShapeDtypeStruct((N, 1), jnp.float32),
            jax.ShapeDtypeStruct((N, 1), jnp.float32),
        ],
    )(x, W1, b1, W2, b2, hp2)


_CMB_B = 2000


def _comb_body(p_ref, g_ref, d2_ref, o_ref):
    o_ref[...] = d2_ref[...] * (p_ref[0] + p_ref[1] + g_ref[...])


def _comb_call(P, g, d2):
    nblk = N // _CMB_B
    return pl.pallas_call(
        _comb_body,
        grid=(nblk,),
        in_specs=[
            pl.BlockSpec((NC, _CMB_B, D), lambda i: (0, i, 0)),
            pl.BlockSpec((_CMB_B, D), lambda i: (i, 0)),
            pl.BlockSpec((_CMB_B, 1), lambda i: (i, 0)),
        ],
        out_specs=pl.BlockSpec((_CMB_B, D), lambda i: (i, 0)),
        out_shape=jax.ShapeDtypeStruct((N, D), jnp.float32),
    )(P, g, d2)


_POOL_B = 2000


def _pool_body(*refs):
    g_refs = refs[:HOP + 1]
    r_ref, s_ref, o_ref = refs[HOP + 1:]
    r = r_ref[...]
    sv = s_ref[...]
    acc = jnp.zeros((_POOL_B, D), jnp.float32)
    for k in range(HOP + 1):
        hk = g_refs[k][...] * r
        z = jnp.dot(hk, sv, preferred_element_type=jnp.float32)
        acc = acc + jax.nn.sigmoid(z) * hk
    o_ref[...] = acc


def _pool_call(gs, r, s):
    nblk = N // _POOL_B
    in_specs = [pl.BlockSpec((_POOL_B, D), lambda i: (i, 0))
                for _ in range(HOP + 1)]
    in_specs.append(pl.BlockSpec((_POOL_B, 1), lambda i: (i, 0)))
    in_specs.append(pl.BlockSpec((D, 1), lambda i: (0, 0)))
    return pl.pallas_call(
        _pool_body,
        grid=(nblk,),
        in_specs=in_specs,
        out_specs=pl.BlockSpec((_POOL_B, D), lambda i: (i, 0)),
        out_shape=jax.ShapeDtypeStruct((N, D), jnp.float32),
    )(*gs, r, s)


def kernel(x, edge_index, W1, b1, W2, b2, s):
    src = edge_index[0].astype(jnp.int32).reshape(NW, NCHUNK, CHUNK)
    dst = edge_index[1].astype(jnp.int32).reshape(NW, NCHUNK, CHUNK)
    zh = jnp.zeros((RPT,), jnp.float32)
    zv = jnp.zeros((ZROWS, D), jnp.float32)

    hp = _deg_kernel(dst, zh)
    hp2 = jnp.transpose(hp[:, :N])

    g, d2, r = _mlp_call(x, W1, b1.reshape(1, DIN), W2, b2.reshape(1, D), hp2)

    gs = [g]
    for _ in range(HOP):
        P = _hop_kernel(src, dst, g, zv)
        g = _comb_call(P, g, d2)
        gs.append(g)

    return _pool_call(gs, r, s)


# speedup vs baseline: 28.8694x; 1.2445x over previous
import functools

import jax
import jax.numpy as jnp
from jax import lax
from jax.experimental import pallas as pl
from jax.experimental.pallas import tpu as pltpu
from jax.experimental.pallas import tpu_sc as plsc

N = 10000
E = 320000
HOP = 10
D = 64
DIN = 128
NC = 2
NS = 16
NW = NC * NS
EPW = E // NW
CHUNK = 125
NCHUNK = EPW // CHUNK
RING = 5
NGRP = NCHUNK // RING
NPAD = 10000
RPT = NPAD // NS
ZROWS = 125
NPADD = 10240
RPTD = NPADD // NS

_mesh = plsc.VectorSubcoreMesh(core_axis_name="c", subcore_axis_name="s")
_sc_params = pltpu.CompilerParams(use_tc_tiling_on_sc=False)


@functools.partial(
    pl.kernel,
    out_type=jax.ShapeDtypeStruct((NC, NPADD), jnp.float32),
    mesh=_mesh,
    compiler_params=_sc_params,
    scratch_types=[
        pltpu.VMEM_SHARED((NPADD,), jnp.float32),
        pltpu.VMEM((NCHUNK, CHUNK), jnp.int32),
        pltpu.VMEM((CHUNK,), jnp.float32),
        pltpu.VMEM((RPTD,), jnp.float32),
        pltpu.SemaphoreType.DMA,
    ],
)
def _deg_kernel(dst_hbm, zh_hbm, out_hbm, acc, didx, ones, zbuf, sem):
    c = lax.axis_index("c")
    s = lax.axis_index("s")
    wid = c * NS + s
    pltpu.sync_copy(dst_hbm.at[wid], didx)
    pltpu.sync_copy(zh_hbm, zbuf)
    for i in range(CHUNK // 16):
        ones[pl.ds(i * 16, 16)] = jnp.ones((16,), jnp.float32)
    ones[pl.ds(CHUNK - 16, 16)] = jnp.ones((16,), jnp.float32)
    pltpu.sync_copy(zbuf, acc.at[pl.ds(s * RPTD, RPTD)])
    plsc.subcore_barrier()

    def grp(t, carry):
        for b in range(RING):
            pltpu.async_copy(ones, acc.at[didx.at[t * RING + b]], sem,
                             add=True)
        for b in range(RING):
            pltpu.make_async_copy(ones, acc.at[didx.at[t * RING + b]],
                                  sem).wait()
        return carry

    lax.fori_loop(0, NGRP, grp, 0)
    plsc.subcore_barrier()
    pltpu.sync_copy(acc.at[pl.ds(s * RPTD, RPTD)],
                    out_hbm.at[c, pl.ds(s * RPTD, RPTD)])


@functools.partial(
    pl.kernel,
    out_type=jax.ShapeDtypeStruct((NC, NPAD, D), jnp.float32),
    mesh=_mesh,
    compiler_params=_sc_params,
    scratch_types=[
        pltpu.VMEM_SHARED((NPAD, D), jnp.float32),
        pltpu.VMEM((NCHUNK, CHUNK), jnp.int32),
        pltpu.VMEM((NCHUNK, CHUNK), jnp.int32),
        pltpu.VMEM((RING, CHUNK, D), jnp.float32),
        pltpu.VMEM((ZROWS, D), jnp.float32),
    ] + [pltpu.SemaphoreType.DMA] * RING,
)
def _hop_kernel(src_hbm, dst_hbm, g_hbm, zv_hbm, out_hbm,
                acc, sidx, didx, rows, zbuf, *sems):
    c = lax.axis_index("c")
    s = lax.axis_index("s")
    wid = c * NS + s
    pltpu.sync_copy(src_hbm.at[wid], sidx)
    pltpu.sync_copy(dst_hbm.at[wid], didx)
    pltpu.sync_copy(zv_hbm, zbuf)
    for b in range(RPT // ZROWS):
        pltpu.sync_copy(zbuf, acc.at[pl.ds(s * RPT + b * ZROWS, ZROWS)])
    plsc.subcore_barrier()

    def gather(j, b):
        pltpu.async_copy(g_hbm.at[sidx.at[j]], rows.at[b], sems[b])

    def wait_gather(j, b):
        pltpu.make_async_copy(g_hbm.at[sidx.at[j]], rows.at[b],
                              sems[b]).wait()

    def scatter(j, b):
        pltpu.async_copy(rows.at[b], acc.at[didx.at[j]], sems[b], add=True)

    def wait_scatter(j, b):
        pltpu.make_async_copy(rows.at[b], acc.at[didx.at[j]],
                              sems[b]).wait()

    for b in range(RING):
        gather(b, b)

    def grp(t, carry):
        j0 = t * RING
        for b in range(RING):
            wait_gather(j0 + b, b)
            scatter(j0 + b, b)
        for b in range(RING):
            wait_scatter(j0 + b, b)
            @pl.when(t < NGRP - 1)
            def _():
                gather(j0 + RING + b, b)
        return carry

    lax.fori_loop(0, NGRP, grp, 0)
    plsc.subcore_barrier()
    pltpu.sync_copy(acc.at[pl.ds(s * RPT, RPT)],
                    out_hbm.at[c, pl.ds(s * RPT, RPT)])


_MLP_B = 2000


def _mlp_body(x_ref, w1_ref, b1_ref, w2_ref, b2_ref, hp_ref,
              g0_ref, d2_ref, r_ref):
    deg = hp_ref[...][:, 0:1] + hp_ref[...][:, 1:2] + 1.0
    dinv = lax.rsqrt(deg)
    h = jnp.maximum(
        jnp.dot(x_ref[...], w1_ref[...], preferred_element_type=jnp.float32)
        + b1_ref[...], 0.0)
    h = jnp.dot(h, w2_ref[...], preferred_element_type=jnp.float32) + b2_ref[...]
    g0_ref[...] = h * dinv
    d2_ref[...] = dinv * dinv
    r_ref[...] = deg * dinv


def _mlp_call(x, W1, b1, W2, b2, hp2):
    nblk = N // _MLP_B
    return pl.pallas_call(
        _mlp_body,
        grid=(nblk,),
        in_specs=[
            pl.BlockSpec((_MLP_B, DIN), lambda i: (i, 0)),
            pl.BlockSpec((DIN, DIN), lambda i: (0, 0)),
            pl.BlockSpec((1, DIN), lambda i: (0, 0)),
            pl.BlockSpec((DIN, D), lambda i: (0, 0)),
            pl.BlockSpec((1, D), lambda i: (0, 0)),
            pl.BlockSpec((_MLP_B, 2), lambda i: (i, 0)),
        ],
        out_specs=[
            pl.BlockSpec((_MLP_B, D), lambda i: (i, 0)),
            pl.BlockSpec((_MLP_B, 1), lambda i: (i, 0)),
            pl.BlockSpec((_MLP_B, 1), lambda i: (i, 0)),
        ],
        out_shape=[
            jax.ShapeDtypeStruct((N, D), jnp.float32),
            jax.ShapeDtypeStruct((N, 1), jnp.float32),
            jax.ShapeDtypeStruct((N, 1), jnp.float32),
        ],
    )(x, W1, b1, W2, b2, hp2)


_CMB_B = 2000


def _comb_body(p_ref, g_ref, d2_ref, o_ref):
    o_ref[...] = d2_ref[...] * (p_ref[0] + p_ref[1] + g_ref[...])


def _comb_call(P, g, d2):
    nblk = N // _CMB_B
    return pl.pallas_call(
        _comb_body,
        grid=(nblk,),
        in_specs=[
            pl.BlockSpec((NC, _CMB_B, D), lambda i: (0, i, 0)),
            pl.BlockSpec((_CMB_B, D), lambda i: (i, 0)),
            pl.BlockSpec((_CMB_B, 1), lambda i: (i, 0)),
        ],
        out_specs=pl.BlockSpec((_CMB_B, D), lambda i: (i, 0)),
        out_shape=jax.ShapeDtypeStruct((N, D), jnp.float32),
    )(P, g, d2)


_POOL_B = 2000


def _pool_body(*refs):
    g_refs = refs[:HOP + 1]
    r_ref, s_ref, o_ref = refs[HOP + 1:]
    r = r_ref[...]
    sv = s_ref[...]
    acc = jnp.zeros((_POOL_B, D), jnp.float32)
    for k in range(HOP + 1):
        hk = g_refs[k][...] * r
        z = jnp.dot(hk, sv, preferred_element_type=jnp.float32)
        acc = acc + jax.nn.sigmoid(z) * hk
    o_ref[...] = acc


def _pool_call(gs, r, s):
    nblk = N // _POOL_B
    in_specs = [pl.BlockSpec((_POOL_B, D), lambda i: (i, 0))
                for _ in range(HOP + 1)]
    in_specs.append(pl.BlockSpec((_POOL_B, 1), lambda i: (i, 0)))
    in_specs.append(pl.BlockSpec((D, 1), lambda i: (0, 0)))
    return pl.pallas_call(
        _pool_body,
        grid=(nblk,),
        in_specs=in_specs,
        out_specs=pl.BlockSpec((_POOL_B, D), lambda i: (i, 0)),
        out_shape=jax.ShapeDtypeStruct((N, D), jnp.float32),
    )(*gs, r, s)


def kernel(x, edge_index, W1, b1, W2, b2, s):
    src = edge_index[0].astype(jnp.int32).reshape(NW, NCHUNK, CHUNK)
    dst = edge_index[1].astype(jnp.int32).reshape(NW, NCHUNK, CHUNK)
    zh = jnp.zeros((RPTD,), jnp.float32)
    zv = jnp.zeros((ZROWS, D), jnp.float32)

    hp = _deg_kernel(dst, zh)
    hp2 = jnp.transpose(hp[:, :N])

    g, d2, r = _mlp_call(x, W1, b1.reshape(1, DIN), W2, b2.reshape(1, D), hp2)

    gs = [g]
    for _ in range(HOP):
        P = _hop_kernel(src, dst, g, zv)
        g = _comb_call(P, g, d2)
        gs.append(g)

    return _pool_call(gs, r, s)
